# baseline (speedup 1.0000x reference)
import functools

import jax
import jax.numpy as jnp
from jax import lax
from jax.experimental import pallas as pl
from jax.experimental.pallas import tpu as pltpu
from jax.experimental.pallas import tpu_sc as plsc

N = 10000
E = 160000
NG = 64
NC, NS = 2, 16
W = NC * NS
KB = 128
NB2 = (E // KB) // W
TBASE = W * NB2 * KB
TAIL = (E - TBASE) // W
CP = 624
REM = N - NS * CP
ZR = 208
BN = 1000
f32 = jnp.float32


def _minmax(a):
    def body(a_ref, lo_ref, hi_ref):
        v = a_ref[...]
        lo_ref[...] = jnp.min(v).reshape(1, 1)
        hi_ref[...] = jnp.max(v).reshape(1, 1)
    return pl.pallas_call(
        body,
        out_shape=(jax.ShapeDtypeStruct((1, 1), f32),
                   jax.ShapeDtypeStruct((1, 1), f32)),
    )(a)


def _quantmm(x, lo, hi, qm, Wroot, nchunk, dc):
    n, fin = x.shape
    fout = Wroot.shape[0]
    dpad = nchunk * dc
    grid = (n // BN,)

    def body_wrap(lo_ref, hi_ref, qm_ref, x_ref, wrt_ref, *out_refs):
        lo_v = lo_ref[0, 0]
        hi_v = hi_ref[0, 0]
        qmax = qm_ref[0, 0]
        s = (hi_v - lo_v) / qmax
        zp = jnp.round(-lo_v / s)
        xb = x_ref[...]
        xq = (jnp.clip(jnp.round(xb / s) + zp, 0.0, qmax) - zp) * s
        out_refs[nchunk][...] = lax.dot_general(
            xq, wrt_ref[...], (((1,), (1,)), ((), ())),
            preferred_element_type=f32)
        y = xq
        if fin < dpad:
            y = jnp.concatenate(
                [y, jnp.zeros((BN, dpad - fin), f32)], axis=1)
        for c in range(nchunk):
            out_refs[c][...] = y[:, c * dc:(c + 1) * dc]

    outs = pl.pallas_call(
        body_wrap,
        grid=grid,
        in_specs=[
            pl.BlockSpec((1, 1), lambda i: (0, 0)),
            pl.BlockSpec((1, 1), lambda i: (0, 0)),
            pl.BlockSpec((1, 1), lambda i: (0, 0)),
            pl.BlockSpec((BN, fin), lambda i: (i, 0)),
            pl.BlockSpec((fout, fin), lambda i: (0, 0)),
        ],
        out_specs=(
            [pl.BlockSpec((BN, dc), lambda i: (i, 0))
             for _ in range(nchunk)]
            + [pl.BlockSpec((BN, fout), lambda i: (i, 0))]
        ),
        out_shape=(
            [jax.ShapeDtypeStruct((n, dc), f32) for _ in range(nchunk)]
            + [jax.ShapeDtypeStruct((n, fout), f32)]
        ),
    )(lo, hi, qm, x, Wroot)
    return outs[:nchunk], outs[nchunk]


def _eaquant(ea2d, lo, hi, qm):
    def body(lo_ref, hi_ref, qm_ref, ea_ref, out_ref, loe_ref, hie_ref):
        lo_v = lo_ref[0, 0]
        hi_v = hi_ref[0, 0]
        qmax = qm_ref[0, 0]
        s = (hi_v - lo_v) / qmax
        zp = jnp.round(-lo_v / s)
        e = ea_ref[...]
        eq = (jnp.clip(jnp.round(e / s) + zp, 0.0, qmax) - zp) * s
        out_ref[...] = eq
        loe_ref[...] = jnp.min(eq).reshape(1, 1)
        hie_ref[...] = jnp.max(eq).reshape(1, 1)
    return pl.pallas_call(
        body,
        out_shape=(jax.ShapeDtypeStruct(ea2d.shape, f32),
                   jax.ShapeDtypeStruct((1, 1), f32),
                   jax.ShapeDtypeStruct((1, 1), f32)),
    )(lo, hi, qm, ea2d)


def _combine(parts, root, Wr, br, fin, nchunk, dc):
    n, fout = root.shape
    grid = (n // BN,)

    def body(*refs):
        p_ref, root_ref, wr_ref, br_ref, z_ref, lo_ref, hi_ref = refs
        p = p_ref[...]
        agg = p[0] + p[1]
        agg2 = jnp.concatenate([agg[c] for c in range(nchunk)], axis=1)
        agg2 = agg2[:, :fin]
        zb = (lax.dot_general(agg2, wr_ref[...], (((1,), (1,)), ((), ())),
                              preferred_element_type=f32)
              + br_ref[...]) + root_ref[...]
        zb = jnp.maximum(zb, 0.0)
        z_ref[...] = zb
        mn = jnp.min(zb).reshape(1, 1)
        mx = jnp.max(zb).reshape(1, 1)
        i = pl.program_id(0)

        @pl.when(i == 0)
        def _():
            lo_ref[...] = mn
            hi_ref[...] = mx

        @pl.when(i > 0)
        def _():
            lo_ref[...] = jnp.minimum(lo_ref[...], mn)
            hi_ref[...] = jnp.maximum(hi_ref[...], mx)

    in_specs = [
        pl.BlockSpec((2, nchunk, BN, dc), lambda i: (0, 0, i, 0)),
        pl.BlockSpec((BN, fout), lambda i: (i, 0)),
        pl.BlockSpec((fout, fin), lambda i: (0, 0)),
        pl.BlockSpec((1, fout), lambda i: (0, 0)),
    ]
    args = [parts, root, Wr, br]
    return pl.pallas_call(
        body,
        grid=grid,
        in_specs=in_specs,
        out_specs=[
            pl.BlockSpec((BN, fout), lambda i: (i, 0)),
            pl.BlockSpec((1, 1), lambda i: (0, 0)),
            pl.BlockSpec((1, 1), lambda i: (0, 0)),
        ],
        out_shape=[
            jax.ShapeDtypeStruct((n, fout), f32),
            jax.ShapeDtypeStruct((1, 1), f32),
            jax.ShapeDtypeStruct((1, 1), f32),
        ],
    )(*args)


def _pool(z, batch2d):
    n, fdim = z.shape
    grid = (n // BN,)

    def body(z_ref, b_ref, sums_ref, cnt_ref):
        zb = z_ref[...]
        bb = b_ref[...]
        g = lax.broadcasted_iota(jnp.int32, (BN, NG), 1)
        mask = (bb == g).astype(f32)
        s = lax.dot_general(mask, zb, (((0,), (0,)), ((), ())),
                            preferred_element_type=f32)
        c = lax.dot_general(mask, jnp.ones((BN, 1), f32),
                            (((0,), (0,)), ((), ())),
                            preferred_element_type=f32)
        i = pl.program_id(0)

        @pl.when(i == 0)
        def _():
            sums_ref[...] = s
            cnt_ref[...] = c

        @pl.when(i > 0)
        def _():
            sums_ref[...] += s
            cnt_ref[...] += c

    return pl.pallas_call(
        body,
        grid=grid,
        in_specs=[
            pl.BlockSpec((BN, fdim), lambda i: (i, 0)),
            pl.BlockSpec((BN, 1), lambda i: (i, 0)),
        ],
        out_specs=[
            pl.BlockSpec((NG, fdim), lambda i: (0, 0)),
            pl.BlockSpec((NG, 1), lambda i: (0, 0)),
        ],
        out_shape=[
            jax.ShapeDtypeStruct((NG, fdim), f32),
            jax.ShapeDtypeStruct((NG, 1), f32),
        ],
    )(z, batch2d)


def _mlp(sums, batchcol, loe, hie, qm, Wd, bd, Wout, bout):
    def body(sums_ref, b_ref, loe_ref, hie_ref, qm_ref,
             w0, w1, w2, b0, b1, b2, wo, bo, out_ref):
        bb = b_ref[...]
        g = lax.broadcasted_iota(jnp.int32, (N, NG), 1)
        mask = (bb == g).astype(f32)
        counts = lax.dot_general(mask, jnp.ones((N, 1), f32),
                                 (((0,), (0,)), ((), ())),
                                 preferred_element_type=f32)
        x = sums_ref[...] / jnp.maximum(counts, 1.0)
        le = loe_ref[0, 0]
        he = hie_ref[0, 0]
        qmax = qm_ref[0, 0]
        ws = [w0, w1, w2]
        bs = [b0, b1, b2]
        for j in range(3):
            lo_v = jnp.minimum(jnp.min(x), le)
            hi_v = jnp.maximum(jnp.max(x), he)
            s = (hi_v - lo_v) / qmax
            zp = jnp.round(-lo_v / s)
            xq = (jnp.clip(jnp.round(x / s) + zp, 0.0, qmax) - zp) * s
            x = lax.dot_general(xq, ws[j][...], (((1,), (1,)), ((), ())),
                                preferred_element_type=f32) + bs[j][...]
            x = jnp.maximum(x, 0.0)
        lo_v = jnp.minimum(jnp.min(x), le)
        hi_v = jnp.maximum(jnp.max(x), he)
        s = (hi_v - lo_v) / qmax
        zp = jnp.round(-lo_v / s)
        xq = (jnp.clip(jnp.round(x / s) + zp, 0.0, qmax) - zp) * s
        out_ref[...] = (jnp.sum(xq * wo[...], axis=1, keepdims=True)
                        + bo[0, 0])

    return pl.pallas_call(
        body,
        out_shape=jax.ShapeDtypeStruct((NG, 1), f32),
    )(sums, batchcol, loe, hie, qm, Wd[0], Wd[1], Wd[2],
      bd[0], bd[1], bd[2], Wout, bout)


@functools.lru_cache(maxsize=None)
def _make_scatter2(nchunk, dc):
    mesh = plsc.VectorSubcoreMesh(core_axis_name="c", subcore_axis_name="s")
    nv = dc // 16

    @functools.partial(
        pl.kernel,
        out_type=jax.ShapeDtypeStruct((NC, nchunk, N, dc), f32),
        mesh=mesh,
        compiler_params=pltpu.CompilerParams(use_tc_tiling_on_sc=False),
        scratch_types=(
            [pltpu.VMEM_SHARED((N, dc), f32)]
            + [
                pltpu.VMEM((2, 16), jnp.int32),
                pltpu.VMEM((KB,), jnp.int32),
                pltpu.VMEM((KB,), jnp.int32),
                pltpu.VMEM((KB,), f32),
                pltpu.VMEM((KB, dc), f32),
                pltpu.SemaphoreType.DMA,
            ]
        ),
    )
    def k(*refs):
        tabs = refs[:nchunk]
        (srcp, dstp, eap, offs_hbm, out_hbm,
         acc, offs, sidx, didx, ewv, rows, sem) = refs[nchunk:]
        core = lax.axis_index("c")
        sub = lax.axis_index("s")
        w = sub * NC + core
        r0 = pl.multiple_of(sub * CP, 8)

        pltpu.sync_copy(offs_hbm.at[pl.ds(w, 2)], offs)
        lane = lax.iota(jnp.int32, 16)
        o0 = offs[0, pl.ds(0, 16)][0]
        o1 = offs[1, pl.ds(0, 16)][0]
        a0 = pl.multiple_of((o0 // 8) * 8, 8)
        a1 = pl.multiple_of(((o1 + 7) // 8) * 8, 8)
        nb = (a1 - a0 + KB - 1) // KB

        for chunk in range(nchunk):
            tab = tabs[chunk]

            def zrow(r, carry):
                for f in range(nv):
                    rows[r, pl.ds(f * 16, 16)] = jnp.zeros((16,), f32)
                return carry
            lax.fori_loop(0, KB, zrow, 0)
            for r in range(CP // KB):
                pltpu.sync_copy(rows, acc.at[pl.ds(r0 + r * KB, KB)])
            pltpu.sync_copy(rows.at[pl.ds(0, CP % KB)],
                            acc.at[pl.ds(r0 + (CP // KB) * KB, CP % KB)])

            @pl.when(sub == 0)
            def _():
                pltpu.sync_copy(rows.at[pl.ds(0, REM)],
                                acc.at[pl.ds(NS * CP, REM)])
            plsc.subcore_barrier()

            def batch_body(b, carry):
                sb = jnp.maximum(jnp.minimum(a0 + b * KB, a1 - KB), 0)
                sb = pl.multiple_of(sb, 8)
                lob = jnp.maximum(o0, a0 + b * KB)
                pltpu.sync_copy(srcp.at[pl.ds(sb, KB)], sidx)
                pltpu.async_copy(tab.at[sidx], rows, sem).wait()
                pltpu.sync_copy(dstp.at[pl.ds(sb, KB)], didx)
                pltpu.sync_copy(eap.at[pl.ds(sb, KB)], ewv)

                def grp(g, c2):
                    gi = sb + g * 16 + lane
                    msk = (gi >= lob) & (gi < o1)
                    ew16 = jnp.where(msk, ewv[pl.ds(g * 16, 16)], 0.0)
                    for ln in range(16):
                        sv = ew16[ln]
                        rb = g * 16 + ln
                        for f in range(nv):
                            rows[rb, pl.ds(f * 16, 16)] = (
                                rows[rb, pl.ds(f * 16, 16)] * sv)
                    return c2
                lax.fori_loop(0, KB // 16, grp, 0)
                pltpu.sync_copy(rows, acc.at[didx], add=True)
                return carry
            lax.fori_loop(0, nb, batch_body, 0)

            plsc.subcore_barrier()
            for r in range(CP // ZR):
                sl = pl.ds(r0 + r * ZR, ZR)
                pltpu.sync_copy(acc.at[sl], out_hbm.at[core, chunk, sl])

            @pl.when(sub == 0)
            def _():
                sl = pl.ds(NS * CP, REM)
                pltpu.sync_copy(acc.at[sl], out_hbm.at[core, chunk, sl])
            plsc.subcore_barrier()

    return k


@functools.lru_cache(maxsize=None)
def _make_scatter(nchunk, dc):
    mesh = plsc.VectorSubcoreMesh(core_axis_name="c", subcore_axis_name="s")
    nv = dc // 16

    @functools.partial(
        pl.kernel,
        out_type=jax.ShapeDtypeStruct((NC, nchunk, N, dc), f32),
        mesh=mesh,
        compiler_params=pltpu.CompilerParams(use_tc_tiling_on_sc=False),
        scratch_types=(
            [pltpu.VMEM_SHARED((N, dc), f32)]
            + [
                pltpu.VMEM((NB2, KB), jnp.int32),
                pltpu.VMEM((NB2, KB), jnp.int32),
                pltpu.VMEM((NB2, KB), f32),
                pltpu.VMEM((KB, dc), f32),
                pltpu.VMEM((KB, dc), f32),
                pltpu.VMEM((16,), jnp.int32),
                pltpu.VMEM((16,), jnp.int32),
                pltpu.VMEM((16,), f32),
                pltpu.VMEM((16, dc), f32),
                pltpu.SemaphoreType.DMA,
                pltpu.SemaphoreType.DMA,
            ]
        ),
    )
    def k(*refs):
        tabs = refs[:nchunk]
        (src2, dst2, ew2, out_hbm,
         acc, sidxs, didxs, ews, rowsA, rowsB,
         sidx8, didx8, ewv8, rows8, semA, semB) = refs[nchunk:]
        core = lax.axis_index("c")
        sub = lax.axis_index("s")
        w = sub * NC + core
        r0 = pl.multiple_of(sub * CP, 8)

        pltpu.sync_copy(src2.at[pl.ds(w * NB2, NB2)], sidxs)
        pltpu.sync_copy(dst2.at[pl.ds(w * NB2, NB2)], didxs)
        pltpu.sync_copy(ew2.at[pl.ds(w * NB2, NB2)], ews)
        trow = W * NB2 + w // NS
        tcol = pl.multiple_of((w % NS) * TAIL, 8)
        pltpu.sync_copy(src2.at[trow, pl.ds(tcol, TAIL)],
                        sidx8.at[pl.ds(0, TAIL)])
        pltpu.sync_copy(dst2.at[trow, pl.ds(tcol, TAIL)],
                        didx8.at[pl.ds(0, TAIL)])
        pltpu.sync_copy(ew2.at[trow, pl.ds(tcol, TAIL)],
                        ewv8.at[pl.ds(0, TAIL)])
        lane = lax.iota(jnp.int32, 16)
        m = lane < TAIL
        sidx8[...] = jnp.where(m, sidx8[...], 0)
        didx8[...] = jnp.where(m, didx8[...], 0)
        ewv8[...] = jnp.where(m, ewv8[...], 0.0)

        def mult_scatter(bb, rows):
            def grp(g, carry):
                ew16 = ews[bb, pl.ds(g * 16, 16)]
                for ln in range(16):
                    sv = ew16[ln]
                    rb = g * 16 + ln
                    for f in range(nv):
                        rows[rb, pl.ds(f * 16, 16)] = (
                            rows[rb, pl.ds(f * 16, 16)] * sv)
                return carry
            lax.fori_loop(0, KB // 16, grp, 0)
            pltpu.sync_copy(rows, acc.at[didxs.at[bb]], add=True)

        for chunk in range(nchunk):
            tab = tabs[chunk]

            def gstart(b, rows, sem):
                pltpu.async_copy(tab.at[sidxs.at[b]], rows, sem)

            def gwait(b, rows, sem):
                pltpu.make_async_copy(tab.at[sidxs.at[b]], rows, sem).wait()

            def zrow(r, carry):
                for f in range(nv):
                    rowsA[r, pl.ds(f * 16, 16)] = jnp.zeros((16,), f32)
                return carry
            lax.fori_loop(0, KB, zrow, 0)
            for r in range(CP // KB):
                pltpu.sync_copy(rowsA, acc.at[pl.ds(r0 + r * KB, KB)])
            pltpu.sync_copy(rowsA.at[pl.ds(0, CP % KB)],
                            acc.at[pl.ds(r0 + (CP // KB) * KB, CP % KB)])

            @pl.when(sub == 0)
            def _():
                pltpu.sync_copy(rowsA.at[pl.ds(0, REM)],
                                acc.at[pl.ds(NS * CP, REM)])
            plsc.subcore_barrier()

            gstart(0, rowsA, semA)

            def dbl(i, carry):
                b0 = 2 * i
                gstart(b0 + 1, rowsB, semB)
                gwait(b0, rowsA, semA)
                mult_scatter(b0, rowsA)
                gstart(b0 + 2, rowsA, semA)
                gwait(b0 + 1, rowsB, semB)
                mult_scatter(b0 + 1, rowsB)
                return carry
            lax.fori_loop(0, (NB2 - 1) // 2, dbl, 0)

            gwait(NB2 - 1, rowsA, semA)
            mult_scatter(NB2 - 1, rowsA)

            pltpu.async_copy(tab.at[sidx8], rows8, semB)
            pltpu.make_async_copy(tab.at[sidx8], rows8, semB).wait()
            ew16 = ewv8[...]
            for ln in range(16):
                sv = ew16[ln]
                for f in range(nv):
                    rows8[ln, pl.ds(f * 16, 16)] = (
                        rows8[ln, pl.ds(f * 16, 16)] * sv)
            pltpu.sync_copy(rows8, acc.at[didx8], add=True)

            plsc.subcore_barrier()
            for r in range(CP // ZR):
                sl = pl.ds(r0 + r * ZR, ZR)
                pltpu.sync_copy(acc.at[sl], out_hbm.at[core, chunk, sl])

            @pl.when(sub == 0)
            def _():
                sl = pl.ds(NS * CP, REM)
                pltpu.sync_copy(acc.at[sl], out_hbm.at[core, chunk, sl])
            plsc.subcore_barrier()

    return k


@functools.lru_cache(maxsize=None)
def _make_pool(fdim):
    mesh = plsc.VectorSubcoreMesh(core_axis_name="c", subcore_axis_name="s")
    nv = fdim // 16
    WIN = 64

    @functools.partial(
        pl.kernel,
        out_type=jax.ShapeDtypeStruct((W, 2, fdim), f32),
        mesh=mesh,
        compiler_params=pltpu.CompilerParams(use_tc_tiling_on_sc=False),
        scratch_types=[
            pltpu.VMEM((3, 16), jnp.int32),
            pltpu.VMEM((WIN, fdim), f32),
            pltpu.VMEM((2, fdim), f32),
        ],
    )
    def k(x_hbm, pb_hbm, out_hbm, pbv, buf, accb):
        core = lax.axis_index("c")
        sub = lax.axis_index("s")
        w = sub * NC + core
        pltpu.sync_copy(pb_hbm.at[pl.ds(2 * w, 3)], pbv)
        b0 = pbv[0, pl.ds(0, 16)][0]
        b1 = pbv[1, pl.ds(0, 16)][0]
        b2 = pbv[2, pl.ds(0, 16)][0]
        for g2 in range(2):
            for f in range(nv):
                accb[g2, pl.ds(f * 16, 16)] = jnp.zeros((16,), f32)
        a0 = pl.multiple_of((b0 // 8) * 8, 8)
        a1 = pl.multiple_of(((b2 + 7) // 8) * 8, 8)
        nb = (a1 - a0 + WIN - 1) // WIN

        def win(b, carry):
            sb = pl.multiple_of(
                jnp.maximum(jnp.minimum(a0 + b * WIN, a1 - WIN), 0), 8)
            lob = jnp.maximum(b0, a0 + b * WIN)
            pltpu.sync_copy(x_hbm.at[pl.ds(sb, WIN)], buf)

            def row(r, c2):
                gi = sb + r
                m0 = (gi >= lob) & (gi < b1)
                m1 = (gi >= lob) & (gi >= b1) & (gi < b2)

                @pl.when(m0)
                def _():
                    for f in range(nv):
                        accb[0, pl.ds(f * 16, 16)] = (
                            accb[0, pl.ds(f * 16, 16)]
                            + buf[r, pl.ds(f * 16, 16)])

                @pl.when(m1)
                def _():
                    for f in range(nv):
                        accb[1, pl.ds(f * 16, 16)] = (
                            accb[1, pl.ds(f * 16, 16)]
                            + buf[r, pl.ds(f * 16, 16)])
                return c2
            lax.fori_loop(0, WIN, row, 0)
            return carry
        lax.fori_loop(0, nb, win, 0)
        pltpu.sync_copy(accb, out_hbm.at[w])

    return k


GCN = [5, 32, 128, 256, 512, 512, 256, 256]
MLPDIMS = [256, 256, 128, 64]


def kernel(x, edge_index, edge_attr, batch, bit_width,
           Wrel, brel, Wroot, Wd, bd, Wout, bout):
    qm = (jnp.float32(2.0) ** bit_width - 1.0).reshape(1, 1)
    src = edge_index[0]
    dst = edge_index[1]
    perm = jnp.argsort(dst, stable=True)
    srcp = src[perm]
    dstp = dst[perm]
    ea2d = edge_attr[perm].reshape(1250, 128)
    bounds = (jnp.arange(W + 1, dtype=jnp.int32) * N) // W
    offs = jnp.searchsorted(dstp, bounds, side='left').astype(jnp.int32)
    offs = jnp.tile(jnp.pad(offs, (0, 40 - (W + 1)))[:, None], (1, 16))
    lo_x, hi_x = _minmax(x)
    lo_e, hi_e = _minmax(ea2d)

    for i in range(7):
        fin, fout = GCN[i], GCN[i + 1]
        dpad = max(16, fin)
        dc = min(dpad, 128)
        nchunk = dpad // dc
        lo = jnp.minimum(lo_x, lo_e)
        hi = jnp.maximum(hi_x, hi_e)
        tabs, root = _quantmm(x, lo, hi, qm, Wroot[i], nchunk, dc)
        ea2d, lo_e, hi_e = _eaquant(ea2d, lo, hi, qm)
        parts = _make_scatter2(nchunk, dc)(*tabs, srcp, dstp,
                                           ea2d.reshape(E), offs)
        x, lo_x, hi_x = _combine(parts, root, Wrel[i],
                                 brel[i].reshape(1, fout), fin, nchunk, dc)

    pb = jnp.searchsorted(batch,
                          jnp.arange(NG + 1, dtype=jnp.int32)).astype(
                              jnp.int32)
    pbrep = jnp.tile(jnp.pad(pb, (0, 72 - (NG + 1)))[:, None], (1, 16))
    sums = jax.ops.segment_sum(x, batch, num_segments=NG)
    out = _mlp(sums, batch.reshape(N, 1), lo_e, hi_e, qm,
               [w for w in Wd], [b.reshape(1, -1) for b in bd],
               Wout, bout.reshape(1, 1))
    return out
